# baseline (speedup 1.0000x reference)
import jax
import jax.numpy as jnp
from jax import lax
from jax.experimental import pallas as pl
from jax.experimental.pallas import tpu as pltpu
from jax.experimental.pallas import tpu_sc as plsc

M = 2097152
B = 1048576
D = 28
DP = 32

NC = 2
NS = 16
L = 16

HALF = M // NC
RM = 49152
NPM = 21
RT = HALF - NPM * RM
SPT = B // NS
SUB = 4096
NSUB = SPT // SUB
CAP = 2048
Q = 128
TRASH = CAP + Q
PADROWS = L
ZW = 4096


def _body(val_hbm, rew_hbm, idx_hbm,
          out_data, out_tr, out_nv,
          acc_data, acc_tr, acc_nv,
          idx_blk, rew_blk, zbuf,
          sel_lidx, sel_sid, sel_rew, sel_one,
          rows, gsem):
    c = lax.axis_index("c")
    s = lax.axis_index("s")
    iota = lax.iota(jnp.int32, L)

    def zf_body(i, _):
        zbuf[i, pl.ds(0, L)] = jnp.zeros((L,), jnp.float32)
        zbuf[i, pl.ds(L, L)] = jnp.zeros((L,), jnp.float32)
        return 0
    lax.fori_loop(0, ZW // DP, zf_body, 0)

    def zi_body(i, _):
        idx_blk[pl.ds(i * L, L)] = jnp.zeros((L,), jnp.int32)
        rew_blk[pl.ds(i * L, L)] = jnp.zeros((L,), jnp.float32)
        return 0
    lax.fori_loop(0, SUB // L, zi_body, 0)

    def do_pass(base, r_chunk, slc):
        def zrow_body(z, _):
            pltpu.sync_copy(
                zbuf,
                acc_data.at[pl.ds(s * slc + z * (ZW // DP), ZW // DP), :])
            return 0
        lax.fori_loop(0, (slc * DP) // ZW, zrow_body, 0)
        pltpu.sync_copy(rew_blk.at[pl.ds(0, slc)],
                        acc_tr.at[pl.ds(s * slc, slc)])
        pltpu.sync_copy(idx_blk.at[pl.ds(0, slc)],
                        acc_nv.at[pl.ds(s * slc, slc)])

        plsc.subcore_barrier()

        def sub_body(b, n):
            row = s * NSUB + b
            pltpu.sync_copy(idx_hbm.at[row], idx_blk)
            pltpu.sync_copy(rew_hbm.at[row], rew_blk)

            def vec_body(i, n):
                iv = idx_blk[pl.ds(i * L, L)]
                lv = iv - base
                mi = (jnp.right_shift(lv, 31) + 1) * \
                     (jnp.right_shift(r_chunk - 1 - lv, 31) + 1)
                hits = jnp.sum(mi, axis=0)

                @pl.when(hits > 0)
                def _():
                    rv = rew_blk[pl.ds(i * L, L)]
                    cs = plsc.cumsum(mi)
                    pos = mi * (n + cs - 1) + (1 - mi) * (TRASH + iota)
                    sid = row * SUB + i * L + iota
                    plsc.store_scatter(sel_lidx, [pos], lv)
                    plsc.store_scatter(sel_sid, [pos], sid)
                    plsc.store_scatter(sel_rew, [pos], rv)
                    plsc.store_scatter(sel_one, [pos], mi)

                return n + hits

            return lax.fori_loop(0, SUB // L, vec_body, n)

        n = lax.fori_loop(0, NSUB, sub_body, jnp.int32(0))

        lax.fori_loop(0, SUB // L, zi_body, 0)

        for j in range(Q // L):
            sel_lidx[pl.ds(n + j * L, L)] = r_chunk + iota
            sel_sid[pl.ds(n + j * L, L)] = j * L + iota
            sel_rew[pl.ds(n + j * L, L)] = jnp.zeros((L,), jnp.float32)
            sel_one[pl.ds(n + j * L, L)] = jnp.zeros((L,), jnp.int32)

        nch = (n + Q - 1) // Q

        def chunk_body(k, _):
            o = k * Q
            descs = []
            for g in range(Q // L):
                siv = sel_sid[pl.ds(o + g * L, L)]
                descs.append(pltpu.async_copy(
                    val_hbm.at[siv], rows.at[pl.ds(g * L, L), :], gsem))
            for dsc in descs:
                dsc.wait()

            def scale_body(g, _):
                rg = sel_rew[pl.ds(o + g * L, L)]
                for r in range(L):
                    gain = rg[r]
                    rr = g * L + r
                    rows[rr, pl.ds(0, L)] = rows[rr, pl.ds(0, L)] * gain
                    rows[rr, pl.ds(L, L)] = rows[rr, pl.ds(L, L)] * gain
                return 0
            lax.fori_loop(0, Q // L, scale_body, 0)

            for g in range(Q // L):
                liv = sel_lidx[pl.ds(o + g * L, L)]
                pltpu.sync_copy(rows.at[pl.ds(g * L, L), :],
                                acc_data.at[liv], add=True)
                pltpu.sync_copy(sel_rew.at[pl.ds(o + g * L, L)],
                                acc_tr.at[liv], add=True)
                pltpu.sync_copy(sel_one.at[pl.ds(o + g * L, L)],
                                acc_nv.at[liv], add=True)
            return 0

        lax.fori_loop(0, nch, chunk_body, 0)

        plsc.subcore_barrier()

        pltpu.sync_copy(acc_data.at[pl.ds(s * slc, slc), :],
                        out_data.at[pl.ds(base + s * slc, slc), :])
        pltpu.sync_copy(acc_tr.at[pl.ds(s * slc, slc)],
                        out_tr.at[pl.ds(base + s * slc, slc)])
        pltpu.sync_copy(acc_nv.at[pl.ds(s * slc, slc)],
                        out_nv.at[pl.ds(base + s * slc, slc)])
        return 0

    def main_pass(p, carry):
        do_pass(c * HALF + p * RM, RM, RM // NS)
        return carry

    lax.fori_loop(0, NPM, main_pass, jnp.int32(0))
    do_pass(c * HALF + NPM * RM, RT, RT // NS)


@jax.jit
def kernel(mem, val, reward, total_reward, idx):
    mesh = plsc.VectorSubcoreMesh(core_axis_name="c", subcore_axis_name="s",
                                  num_cores=NC, num_subcores=NS)
    f = pl.kernel(
        _body,
        out_type=(
            jax.ShapeDtypeStruct((M, DP), jnp.float32),
            jax.ShapeDtypeStruct((M,), jnp.float32),
            jax.ShapeDtypeStruct((M,), jnp.int32),
        ),
        mesh=mesh,
        scratch_types=(
            pltpu.VMEM_SHARED((RM + PADROWS, DP), jnp.float32),
            pltpu.VMEM_SHARED((RM + PADROWS,), jnp.float32),
            pltpu.VMEM_SHARED((RM + PADROWS,), jnp.int32),
            pltpu.VMEM((SUB,), jnp.int32),
            pltpu.VMEM((SUB,), jnp.float32),
            pltpu.VMEM((ZW // DP, DP), jnp.float32),
            pltpu.VMEM((TRASH + L,), jnp.int32),
            pltpu.VMEM((TRASH + L,), jnp.int32),
            pltpu.VMEM((TRASH + L,), jnp.float32),
            pltpu.VMEM((TRASH + L,), jnp.int32),
            pltpu.VMEM((Q, DP), jnp.float32),
            pltpu.SemaphoreType.DMA,
        ),
        name="mcts_scatter_add",
        compiler_params=pltpu.CompilerParams(needs_layout_passes=False,
                                             use_tc_tiling_on_sc=False),
    )
    val32 = jnp.pad(val, ((0, 0), (0, DP - D)))
    d32, d_tr, nv = f(val32, reward.reshape(B // SUB, SUB),
                      idx.reshape(B // SUB, SUB))
    return mem + d32[:, :D], total_reward + d_tr, nv


# speedup vs baseline: 1.1890x; 1.1890x over previous
import jax
import jax.numpy as jnp
from jax import lax
from jax.experimental import pallas as pl
from jax.experimental.pallas import tpu as pltpu
from jax.experimental.pallas import tpu_sc as plsc

M = 2097152
B = 1048576
D = 28
DP = 32

NC = 2
NS = 16
L = 16

HALF = M // NC
RM = 49152
NPM = 21
RT = HALF - NPM * RM
SPT = B // NS
SUB = 4096
NSUB = SPT // SUB
CAP = 2048
Q = 128
TRASH = CAP + Q
PADROWS = L
ZW = 4096


def _body(val_hbm, rew_hbm, idx_hbm,
          out_data, out_tr, out_nv,
          acc_data, acc_tr, acc_nv,
          idx_blk, rew_blk, zbuf,
          sel_lidx, sel_sid, sel_rew, sel_one,
          rows, gsem):
    c = lax.axis_index("c")
    s = lax.axis_index("s")
    iota = lax.iota(jnp.int32, L)

    def zf_body(i, _):
        zbuf[i, pl.ds(0, L)] = jnp.zeros((L,), jnp.float32)
        zbuf[i, pl.ds(L, L)] = jnp.zeros((L,), jnp.float32)
        return 0
    lax.fori_loop(0, ZW // DP, zf_body, 0)

    def zi_body(i, _):
        idx_blk[pl.ds(i * L, L)] = jnp.zeros((L,), jnp.int32)
        rew_blk[pl.ds(i * L, L)] = jnp.zeros((L,), jnp.float32)
        return 0
    lax.fori_loop(0, SUB // L, zi_body, 0)

    def do_pass(base, r_chunk, slc):
        def zrow_body(z, _):
            pltpu.sync_copy(
                zbuf,
                acc_data.at[pl.ds(s * slc + z * (ZW // DP), ZW // DP), :])
            return 0
        lax.fori_loop(0, (slc * DP) // ZW, zrow_body, 0)
        pltpu.sync_copy(rew_blk.at[pl.ds(0, slc)],
                        acc_tr.at[pl.ds(s * slc, slc)])
        pltpu.sync_copy(idx_blk.at[pl.ds(0, slc)],
                        acc_nv.at[pl.ds(s * slc, slc)])

        plsc.subcore_barrier()

        def sub_body(b, n):
            row = s * NSUB + b
            pltpu.sync_copy(idx_hbm.at[row], idx_blk)
            pltpu.sync_copy(rew_hbm.at[row], rew_blk)

            def vec_body(i, n):
                iv = idx_blk[pl.ds(i * L, L)]
                lv = iv - base
                mi = (jnp.right_shift(lv, 31) + 1) * \
                     (jnp.right_shift(r_chunk - 1 - lv, 31) + 1)
                rv = rew_blk[pl.ds(i * L, L)]
                cs = plsc.cumsum(mi)
                pos = mi * (n + cs - 1) + (1 - mi) * (TRASH + iota)
                sid = row * SUB + i * L + iota
                plsc.store_scatter(sel_lidx, [pos], lv)
                plsc.store_scatter(sel_sid, [pos], sid)
                plsc.store_scatter(sel_rew, [pos], rv)
                plsc.store_scatter(sel_one, [pos], mi)
                return n + cs[15]

            return lax.fori_loop(0, SUB // L, vec_body, n)

        n = lax.fori_loop(0, NSUB, sub_body, jnp.int32(0))

        lax.fori_loop(0, SUB // L, zi_body, 0)

        for j in range(Q // L):
            sel_lidx[pl.ds(n + j * L, L)] = r_chunk + iota
            sel_sid[pl.ds(n + j * L, L)] = j * L + iota
            sel_rew[pl.ds(n + j * L, L)] = jnp.zeros((L,), jnp.float32)
            sel_one[pl.ds(n + j * L, L)] = jnp.zeros((L,), jnp.int32)

        nch = (n + Q - 1) // Q

        def chunk_body(k, _):
            o = k * Q
            descs = []
            for g in range(Q // L):
                siv = sel_sid[pl.ds(o + g * L, L)]
                descs.append(pltpu.async_copy(
                    val_hbm.at[siv], rows.at[pl.ds(g * L, L), :], gsem))
            for dsc in descs:
                dsc.wait()

            def scale_body(g, _):
                rg = sel_rew[pl.ds(o + g * L, L)]
                for r in range(L):
                    gain = rg[r]
                    rr = g * L + r
                    rows[rr, pl.ds(0, L)] = rows[rr, pl.ds(0, L)] * gain
                    rows[rr, pl.ds(L, L)] = rows[rr, pl.ds(L, L)] * gain
                return 0
            lax.fori_loop(0, Q // L, scale_body, 0)

            for g in range(Q // L):
                liv = sel_lidx[pl.ds(o + g * L, L)]
                pltpu.sync_copy(rows.at[pl.ds(g * L, L), :],
                                acc_data.at[liv], add=True)
                pltpu.sync_copy(sel_rew.at[pl.ds(o + g * L, L)],
                                acc_tr.at[liv], add=True)
                pltpu.sync_copy(sel_one.at[pl.ds(o + g * L, L)],
                                acc_nv.at[liv], add=True)
            return 0

        lax.fori_loop(0, nch, chunk_body, 0)

        plsc.subcore_barrier()

        pltpu.sync_copy(acc_data.at[pl.ds(s * slc, slc), :],
                        out_data.at[pl.ds(base + s * slc, slc), :])
        pltpu.sync_copy(acc_tr.at[pl.ds(s * slc, slc)],
                        out_tr.at[pl.ds(base + s * slc, slc)])
        pltpu.sync_copy(acc_nv.at[pl.ds(s * slc, slc)],
                        out_nv.at[pl.ds(base + s * slc, slc)])
        return 0

    def main_pass(p, carry):
        do_pass(c * HALF + p * RM, RM, RM // NS)
        return carry

    lax.fori_loop(0, NPM, main_pass, jnp.int32(0))
    do_pass(c * HALF + NPM * RM, RT, RT // NS)


@jax.jit
def kernel(mem, val, reward, total_reward, idx):
    mesh = plsc.VectorSubcoreMesh(core_axis_name="c", subcore_axis_name="s",
                                  num_cores=NC, num_subcores=NS)
    f = pl.kernel(
        _body,
        out_type=(
            jax.ShapeDtypeStruct((M, DP), jnp.float32),
            jax.ShapeDtypeStruct((M,), jnp.float32),
            jax.ShapeDtypeStruct((M,), jnp.int32),
        ),
        mesh=mesh,
        scratch_types=(
            pltpu.VMEM_SHARED((RM + PADROWS, DP), jnp.float32),
            pltpu.VMEM_SHARED((RM + PADROWS,), jnp.float32),
            pltpu.VMEM_SHARED((RM + PADROWS,), jnp.int32),
            pltpu.VMEM((SUB,), jnp.int32),
            pltpu.VMEM((SUB,), jnp.float32),
            pltpu.VMEM((ZW // DP, DP), jnp.float32),
            pltpu.VMEM((TRASH + L,), jnp.int32),
            pltpu.VMEM((TRASH + L,), jnp.int32),
            pltpu.VMEM((TRASH + L,), jnp.float32),
            pltpu.VMEM((TRASH + L,), jnp.int32),
            pltpu.VMEM((Q, DP), jnp.float32),
            pltpu.SemaphoreType.DMA,
        ),
        name="mcts_scatter_add",
        compiler_params=pltpu.CompilerParams(needs_layout_passes=False,
                                             use_tc_tiling_on_sc=False),
    )
    val32 = jnp.pad(val, ((0, 0), (0, DP - D)))
    d32, d_tr, nv = f(val32, reward.reshape(B // SUB, SUB),
                      idx.reshape(B // SUB, SUB))
    return mem + d32[:, :D], total_reward + d_tr, nv
